# baseline (speedup 1.0000x reference)
import jax
import jax.numpy as jnp
from jax import lax
from jax.experimental import pallas as pl
from jax.experimental.pallas import tpu as pltpu
from jax.experimental.pallas import tpu_sc as plsc

B, N = 64, 8192
LANES = 16
NUM_CORES = 2
NUM_SUBCORES = 16
NUM_WORKERS = NUM_CORES * NUM_SUBCORES
ROWS_PER_WORKER = B // NUM_WORKERS
NBLK = N // LANES


def _softplus_body(score_ref, out_ref):
    s = score_ref[...]
    sp = jnp.maximum(s, 0.0) + jnp.log1p(jnp.exp(-jnp.abs(s)))
    out_ref[0, 0] = jnp.sum(sp)


def _softplus_sum(score):
    return pl.pallas_call(
        _softplus_body,
        out_shape=jax.ShapeDtypeStruct((1, 1), jnp.float32),
        out_specs=pl.BlockSpec(memory_space=pltpu.SMEM),
    )(score)


def _shuffle(x, idx):
    dnums = lax.GatherDimensionNumbers(
        offset_dims=(), collapsed_slice_dims=(0,), start_index_map=(0,)
    )
    return lax.gather(
        x,
        idx[:, None],
        dnums,
        (1,),
        mode=lax.GatherScatterMode.PROMISE_IN_BOUNDS,
    )


def _bfly_min(x):
    lane = lax.iota(jnp.int32, LANES)
    for sh in (8, 4, 2, 1):
        x = jnp.minimum(x, _shuffle(x, lane ^ sh))
    return x


def _bfly_sum(x):
    lane = lax.iota(jnp.int32, LANES)
    for sh in (8, 4, 2, 1):
        x = x + _shuffle(x, lane ^ sh)
    return x


def _insert(state, x, i):
    a0, a1, a2, a3, i0, i1, i2, i3 = state
    c = x < a3
    a3 = jnp.where(c, x, a3)
    i3 = jnp.where(c, i, i3)
    c = a3 < a2
    a2, a3 = jnp.where(c, a3, a2), jnp.where(c, a2, a3)
    i2, i3 = jnp.where(c, i3, i2), jnp.where(c, i2, i3)
    c = a2 < a1
    a1, a2 = jnp.where(c, a2, a1), jnp.where(c, a1, a2)
    i1, i2 = jnp.where(c, i2, i1), jnp.where(c, i1, i2)
    c = a1 < a0
    a0, a1 = jnp.where(c, a1, a0), jnp.where(c, a0, a1)
    i0, i1 = jnp.where(c, i1, i0), jnp.where(c, i0, i1)
    return (a0, a1, a2, a3, i0, i1, i2, i3)


def _merge_and_gather(state, sco_v):
    lane = lax.iota(jnp.int32, LANES)
    inf = jnp.float32(jnp.inf)

    def round_k(k, carry):
        a0, a1, a2, a3, i0, i1, i2, i3, idxs = carry
        tv = _bfly_min(a0)
        ti = _bfly_min(jnp.where(a0 == tv, i0, N))
        idxs = jnp.where(lane == k, ti, idxs)
        pop = (a0 == tv) & (i0 == ti)
        a0 = jnp.where(pop, a1, a0)
        i0 = jnp.where(pop, i1, i0)
        a1 = jnp.where(pop, a2, a1)
        i1 = jnp.where(pop, i2, i1)
        a2 = jnp.where(pop, a3, a2)
        i2 = jnp.where(pop, i3, i2)
        a3 = jnp.where(pop, inf, a3)
        i3 = jnp.where(pop, N, i3)
        return (a0, a1, a2, a3, i0, i1, i2, i3, idxs)

    idxs0 = jnp.zeros((LANES,), jnp.int32)
    out = lax.fori_loop(0, 4, round_k, state + (idxs0,))
    g = plsc.load_gather(sco_v, [out[-1]])
    return _bfly_sum(jnp.where(lane < 4, g, 0.0))


def _sc_body(
    score_hbm, error_hbm, out_hbm, e0, e1, s0, s1, res_v, sm0, sm1, sm2, sm3
):
    wid = lax.axis_index("s") * NUM_CORES + lax.axis_index("c")
    r0 = wid * ROWS_PER_WORKER
    r1 = r0 + 1
    he0 = pltpu.async_copy(error_hbm.at[r0], e0, sm0)
    he1 = pltpu.async_copy(error_hbm.at[r1], e1, sm1)
    hs0 = pltpu.async_copy(score_hbm.at[r0], s0, sm2)
    hs1 = pltpu.async_copy(score_hbm.at[r1], s1, sm3)
    he0.wait()
    he1.wait()

    lane = lax.iota(jnp.int32, LANES)
    inf = jnp.float32(jnp.inf)
    va = jnp.full((LANES,), inf, jnp.float32)
    vi = jnp.full((LANES,), N, jnp.int32)
    init = (va, va, va, va, vi, vi, vi, vi)

    def body(j, carry):
        st0, st1 = carry
        base = j * LANES
        i = base + lane
        st0 = _insert(st0, e0[pl.ds(base, LANES)], i)
        st1 = _insert(st1, e1[pl.ds(base, LANES)], i)
        return st0, st1

    st0, st1 = lax.fori_loop(0, NBLK, body, (init, init))

    hs0.wait()
    hs1.wait()
    sum0 = _merge_and_gather(st0, s0)
    sum1 = _merge_and_gather(st1, s1)
    res = jnp.where(lane == 0, sum0, jnp.where(lane == 1, sum1, 0.0))
    res_v[...] = res
    pltpu.sync_copy(res_v, out_hbm.at[wid])


def _selected_sums(score, error):
    mesh = plsc.VectorSubcoreMesh(
        core_axis_name="c",
        subcore_axis_name="s",
        num_cores=NUM_CORES,
        num_subcores=NUM_SUBCORES,
    )
    k = pl.kernel(
        _sc_body,
        out_type=jax.ShapeDtypeStruct((NUM_WORKERS, LANES), jnp.float32),
        mesh=mesh,
        scratch_types=[
            pltpu.VMEM((N,), jnp.float32),
            pltpu.VMEM((N,), jnp.float32),
            pltpu.VMEM((N,), jnp.float32),
            pltpu.VMEM((N,), jnp.float32),
            pltpu.VMEM((LANES,), jnp.float32),
            pltpu.SemaphoreType.DMA,
            pltpu.SemaphoreType.DMA,
            pltpu.SemaphoreType.DMA,
            pltpu.SemaphoreType.DMA,
        ],
        compiler_params=pltpu.CompilerParams(needs_layout_passes=False),
    )
    return k(score, error)


def kernel(score, error):
    sp_sum = _softplus_sum(score)[0, 0]
    sel = _selected_sums(score, error)
    sel_sum = jnp.sum(sel[:, :ROWS_PER_WORKER])
    return (sp_sum - sel_sum) / (B * N)


# speedup vs baseline: 1.0290x; 1.0290x over previous
import jax
import jax.numpy as jnp
from jax import lax
from jax.experimental import pallas as pl
from jax.experimental.pallas import tpu as pltpu
from jax.experimental.pallas import tpu_sc as plsc

B, N = 64, 8192
LANES = 16
NUM_CORES = 2
NUM_SUBCORES = 16
NUM_WORKERS = NUM_CORES * NUM_SUBCORES
ROWS_PER_WORKER = B // NUM_WORKERS
NBLK = N // LANES


def _softplus_body(score_ref, out_ref):
    s = score_ref[...]
    sp = jnp.maximum(s, 0.0) + jnp.log1p(jnp.exp(-jnp.abs(s)))
    out_ref[0, 0] = jnp.sum(sp)


def _softplus_sum(score):
    return pl.pallas_call(
        _softplus_body,
        out_shape=jax.ShapeDtypeStruct((1, 1), jnp.float32),
        out_specs=pl.BlockSpec(memory_space=pltpu.SMEM),
    )(score)


def _shuffle(x, idx):
    dnums = lax.GatherDimensionNumbers(
        offset_dims=(), collapsed_slice_dims=(0,), start_index_map=(0,)
    )
    return lax.gather(
        x,
        idx[:, None],
        dnums,
        (1,),
        mode=lax.GatherScatterMode.PROMISE_IN_BOUNDS,
    )


def _bfly_min(x):
    lane = lax.iota(jnp.int32, LANES)
    for sh in (8, 4, 2, 1):
        x = jnp.minimum(x, _shuffle(x, lane ^ sh))
    return x


def _bfly_sum(x):
    lane = lax.iota(jnp.int32, LANES)
    for sh in (8, 4, 2, 1):
        x = x + _shuffle(x, lane ^ sh)
    return x


def _insert(state, x, i):
    a0, a1, a2, a3, i0, i1, i2, i3 = state
    c = x < a3
    a3 = jnp.where(c, x, a3)
    i3 = jnp.where(c, i, i3)
    c = a3 < a2
    a2, a3 = jnp.where(c, a3, a2), jnp.where(c, a2, a3)
    i2, i3 = jnp.where(c, i3, i2), jnp.where(c, i2, i3)
    c = a2 < a1
    a1, a2 = jnp.where(c, a2, a1), jnp.where(c, a1, a2)
    i1, i2 = jnp.where(c, i2, i1), jnp.where(c, i1, i2)
    c = a1 < a0
    a0, a1 = jnp.where(c, a1, a0), jnp.where(c, a0, a1)
    i0, i1 = jnp.where(c, i1, i0), jnp.where(c, i0, i1)
    return (a0, a1, a2, a3, i0, i1, i2, i3)


def _merge_and_gather(state, sco_v):
    a0, a1, a2, a3, i0, i1, i2, i3 = state
    lane = lax.iota(jnp.int32, LANES)
    inf = jnp.float32(jnp.inf)
    idxs = jnp.zeros((LANES,), jnp.int32)
    for k in range(4):
        tv = _bfly_min(a0)
        ti = _bfly_min(jnp.where(a0 == tv, i0, N))
        idxs = jnp.where(lane == k, ti, idxs)
        pop = (a0 == tv) & (i0 == ti)
        a0 = jnp.where(pop, a1, a0)
        i0 = jnp.where(pop, i1, i0)
        a1 = jnp.where(pop, a2, a1)
        i1 = jnp.where(pop, i2, i1)
        a2 = jnp.where(pop, a3, a2)
        i2 = jnp.where(pop, i3, i2)
        a3 = jnp.where(pop, inf, a3)
        i3 = jnp.where(pop, N, i3)
    g = plsc.load_gather(sco_v, [idxs])
    return _bfly_sum(jnp.where(lane < 4, g, 0.0))


def _sc_body(
    score_hbm, error_hbm, out_hbm, e0, e1, s0, s1, res_v, sm0, sm1, sm2, sm3
):
    wid = lax.axis_index("s") * NUM_CORES + lax.axis_index("c")
    r0 = wid * ROWS_PER_WORKER
    r1 = r0 + 1
    H = N // 2
    ha0 = pltpu.async_copy(error_hbm.at[r0, pl.ds(0, H)], e0.at[pl.ds(0, H)], sm0)
    ha1 = pltpu.async_copy(error_hbm.at[r1, pl.ds(0, H)], e1.at[pl.ds(0, H)], sm1)
    hb0 = pltpu.async_copy(
        error_hbm.at[r0, pl.ds(H, H)], e0.at[pl.ds(H, H)], sm2
    )
    hb1 = pltpu.async_copy(
        error_hbm.at[r1, pl.ds(H, H)], e1.at[pl.ds(H, H)], sm3
    )
    ha0.wait()
    ha1.wait()
    hs0 = pltpu.async_copy(score_hbm.at[r0], s0, sm0)
    hs1 = pltpu.async_copy(score_hbm.at[r1], s1, sm1)

    lane = lax.iota(jnp.int32, LANES)
    inf = jnp.float32(jnp.inf)
    va = jnp.full((LANES,), inf, jnp.float32)
    vi = jnp.full((LANES,), N, jnp.int32)
    init = (va, va, va, va, vi, vi, vi, vi)

    def body(j, carry):
        st0, st1 = carry
        base = j * LANES
        i = base + lane
        st0 = _insert(st0, e0[pl.ds(base, LANES)], i)
        st1 = _insert(st1, e1[pl.ds(base, LANES)], i)
        return st0, st1

    st0, st1 = lax.fori_loop(0, NBLK // 2, body, (init, init))
    hb0.wait()
    hb1.wait()
    st0, st1 = lax.fori_loop(NBLK // 2, NBLK, body, (st0, st1))

    hs0.wait()
    hs1.wait()
    sum0 = _merge_and_gather(st0, s0)
    sum1 = _merge_and_gather(st1, s1)
    res = jnp.where(lane == 0, sum0, jnp.where(lane == 1, sum1, 0.0))
    res_v[...] = res
    pltpu.sync_copy(res_v, out_hbm.at[wid])


def _selected_sums(score, error):
    mesh = plsc.VectorSubcoreMesh(
        core_axis_name="c",
        subcore_axis_name="s",
        num_cores=NUM_CORES,
        num_subcores=NUM_SUBCORES,
    )
    k = pl.kernel(
        _sc_body,
        out_type=jax.ShapeDtypeStruct((NUM_WORKERS, LANES), jnp.float32),
        mesh=mesh,
        scratch_types=[
            pltpu.VMEM((N,), jnp.float32),
            pltpu.VMEM((N,), jnp.float32),
            pltpu.VMEM((N,), jnp.float32),
            pltpu.VMEM((N,), jnp.float32),
            pltpu.VMEM((LANES,), jnp.float32),
            pltpu.SemaphoreType.DMA,
            pltpu.SemaphoreType.DMA,
            pltpu.SemaphoreType.DMA,
            pltpu.SemaphoreType.DMA,
        ],
        compiler_params=pltpu.CompilerParams(needs_layout_passes=False),
    )
    return k(score, error)


def kernel(score, error):
    sp_sum = _softplus_sum(score)[0, 0]
    sel = _selected_sums(score, error)
    sel_sum = jnp.sum(sel[:, :ROWS_PER_WORKER])
    return (sp_sum - sel_sum) / (B * N)


# speedup vs baseline: 1.0341x; 1.0050x over previous
import jax
import jax.numpy as jnp
from jax import lax
from jax.experimental import pallas as pl
from jax.experimental.pallas import tpu as pltpu
from jax.experimental.pallas import tpu_sc as plsc

B, N = 64, 8192
LANES = 16
NUM_CORES = 2
NUM_SUBCORES = 16
NUM_WORKERS = NUM_CORES * NUM_SUBCORES
ROWS_PER_WORKER = B // NUM_WORKERS
NBLK = N // LANES


def _softplus_body(score_ref, out_ref):
    s = score_ref[...]
    sp = jnp.maximum(s, 0.0) + jnp.log1p(jnp.exp(-jnp.abs(s)))
    out_ref[0, 0] = jnp.sum(sp)


def _softplus_sum(score):
    return pl.pallas_call(
        _softplus_body,
        out_shape=jax.ShapeDtypeStruct((1, 1), jnp.float32),
        out_specs=pl.BlockSpec(memory_space=pltpu.SMEM),
    )(score)


def _shuffle(x, idx):
    dnums = lax.GatherDimensionNumbers(
        offset_dims=(), collapsed_slice_dims=(0,), start_index_map=(0,)
    )
    return lax.gather(
        x,
        idx[:, None],
        dnums,
        (1,),
        mode=lax.GatherScatterMode.PROMISE_IN_BOUNDS,
    )


def _bfly_min(x):
    lane = lax.iota(jnp.int32, LANES)
    for sh in (8, 4, 2, 1):
        x = jnp.minimum(x, _shuffle(x, lane ^ sh))
    return x


def _bfly_sum(x):
    lane = lax.iota(jnp.int32, LANES)
    for sh in (8, 4, 2, 1):
        x = x + _shuffle(x, lane ^ sh)
    return x


def _insert(state, x, i):
    a0, a1, a2, a3, i0, i1, i2, i3 = state
    c = x < a3
    a3 = jnp.where(c, x, a3)
    i3 = jnp.where(c, i, i3)
    c = a3 < a2
    a2, a3 = jnp.where(c, a3, a2), jnp.where(c, a2, a3)
    i2, i3 = jnp.where(c, i3, i2), jnp.where(c, i2, i3)
    c = a2 < a1
    a1, a2 = jnp.where(c, a2, a1), jnp.where(c, a1, a2)
    i1, i2 = jnp.where(c, i2, i1), jnp.where(c, i1, i2)
    c = a1 < a0
    a0, a1 = jnp.where(c, a1, a0), jnp.where(c, a0, a1)
    i0, i1 = jnp.where(c, i1, i0), jnp.where(c, i0, i1)
    return (a0, a1, a2, a3, i0, i1, i2, i3)


def _merge_and_gather(state, sco_v):
    a0, a1, a2, a3, i0, i1, i2, i3 = state
    lane = lax.iota(jnp.int32, LANES)
    inf = jnp.float32(jnp.inf)
    idxs = jnp.zeros((LANES,), jnp.int32)
    for k in range(4):
        tv = _bfly_min(a0)
        ti = _bfly_min(jnp.where(a0 == tv, i0, N))
        idxs = jnp.where(lane == k, ti, idxs)
        pop = (a0 == tv) & (i0 == ti)
        a0 = jnp.where(pop, a1, a0)
        i0 = jnp.where(pop, i1, i0)
        a1 = jnp.where(pop, a2, a1)
        i1 = jnp.where(pop, i2, i1)
        a2 = jnp.where(pop, a3, a2)
        i2 = jnp.where(pop, i3, i2)
        a3 = jnp.where(pop, inf, a3)
        i3 = jnp.where(pop, N, i3)
    g = plsc.load_gather(sco_v, [idxs])
    return _bfly_sum(jnp.where(lane < 4, g, 0.0))


def _sc_body(
    score_hbm, error_hbm, out_hbm, e0, e1, s0, s1, res_v, sm0, sm1, sm2, sm3
):
    wid = lax.axis_index("s") * NUM_CORES + lax.axis_index("c")
    r0 = wid * ROWS_PER_WORKER
    r1 = r0 + 1
    H = N // 2
    ha0 = pltpu.async_copy(error_hbm.at[r0, pl.ds(0, H)], e0.at[pl.ds(0, H)], sm0)
    ha1 = pltpu.async_copy(error_hbm.at[r1, pl.ds(0, H)], e1.at[pl.ds(0, H)], sm1)
    hb0 = pltpu.async_copy(
        error_hbm.at[r0, pl.ds(H, H)], e0.at[pl.ds(H, H)], sm2
    )
    hb1 = pltpu.async_copy(
        error_hbm.at[r1, pl.ds(H, H)], e1.at[pl.ds(H, H)], sm3
    )
    ha0.wait()
    ha1.wait()
    hs0 = pltpu.async_copy(score_hbm.at[r0], s0, sm0)
    hs1 = pltpu.async_copy(score_hbm.at[r1], s1, sm1)

    lane = lax.iota(jnp.int32, LANES)
    inf = jnp.float32(jnp.inf)
    va = jnp.full((LANES,), inf, jnp.float32)
    vi = jnp.full((LANES,), N, jnp.int32)
    init = (va, va, va, va, vi, vi, vi, vi)

    def body(j, carry):
        st0, st1 = carry
        base = j * LANES
        i = base + lane
        st0 = _insert(st0, e0[pl.ds(base, LANES)], i)
        st1 = _insert(st1, e1[pl.ds(base, LANES)], i)
        return st0, st1

    st0, st1 = lax.fori_loop(0, NBLK // 2, body, (init, init))
    hb0.wait()
    hb1.wait()
    st0, st1 = lax.fori_loop(NBLK // 2, NBLK, body, (st0, st1))

    hs0.wait()
    hs1.wait()
    sum0 = _merge_and_gather(st0, s0)
    sum1 = _merge_and_gather(st1, s1)
    res = jnp.where(lane == 0, sum0, jnp.where(lane == 1, sum1, 0.0))
    res_v[...] = res
    pltpu.sync_copy(res_v, out_hbm.at[wid])


def _selected_sums(score, error):
    mesh = plsc.VectorSubcoreMesh(
        core_axis_name="c",
        subcore_axis_name="s",
        num_cores=NUM_CORES,
        num_subcores=NUM_SUBCORES,
    )
    k = pl.kernel(
        _sc_body,
        out_type=jax.ShapeDtypeStruct((NUM_WORKERS, LANES), jnp.float32),
        mesh=mesh,
        scratch_types=[
            pltpu.VMEM((N,), jnp.float32),
            pltpu.VMEM((N,), jnp.float32),
            pltpu.VMEM((N,), jnp.float32),
            pltpu.VMEM((N,), jnp.float32),
            pltpu.VMEM((LANES,), jnp.float32),
            pltpu.SemaphoreType.DMA,
            pltpu.SemaphoreType.DMA,
            pltpu.SemaphoreType.DMA,
            pltpu.SemaphoreType.DMA,
        ],
        compiler_params=pltpu.CompilerParams(
            needs_layout_passes=False, skip_device_barrier=True
        ),
    )
    return k(score, error)


def kernel(score, error):
    sp_sum = _softplus_sum(score)[0, 0]
    sel = _selected_sums(score, error)
    sel_sum = jnp.sum(sel[:, :ROWS_PER_WORKER])
    return (sp_sum - sel_sum) / (B * N)


# speedup vs baseline: 1.0390x; 1.0048x over previous
import jax
import jax.numpy as jnp
from jax import lax
from jax.experimental import pallas as pl
from jax.experimental.pallas import tpu as pltpu
from jax.experimental.pallas import tpu_sc as plsc

B, N = 64, 8192
LANES = 16
NUM_CORES = 2
NUM_SUBCORES = 16
NUM_WORKERS = NUM_CORES * NUM_SUBCORES
ROWS_PER_WORKER = B // NUM_WORKERS
NBLK = N // LANES


def _softplus_body(score_ref, out_ref):
    s = score_ref[...]
    sp = jnp.maximum(s, 0.0) + jnp.log1p(jnp.exp(-jnp.abs(s)))
    out_ref[0, 0] = jnp.sum(sp)


def _softplus_sum(score):
    return pl.pallas_call(
        _softplus_body,
        out_shape=jax.ShapeDtypeStruct((1, 1), jnp.float32),
        out_specs=pl.BlockSpec(memory_space=pltpu.SMEM),
    )(score)


def _shuffle(x, idx):
    dnums = lax.GatherDimensionNumbers(
        offset_dims=(), collapsed_slice_dims=(0,), start_index_map=(0,)
    )
    return lax.gather(
        x,
        idx[:, None],
        dnums,
        (1,),
        mode=lax.GatherScatterMode.PROMISE_IN_BOUNDS,
    )


def _bfly_min(x):
    lane = lax.iota(jnp.int32, LANES)
    for sh in (8, 4, 2, 1):
        x = jnp.minimum(x, _shuffle(x, lane ^ sh))
    return x


def _bfly_sum(x):
    lane = lax.iota(jnp.int32, LANES)
    for sh in (8, 4, 2, 1):
        x = x + _shuffle(x, lane ^ sh)
    return x


def _insert(state, x, i):
    a0, a1, a2, a3, i0, i1, i2, i3 = state
    c = x < a3
    a3 = jnp.where(c, x, a3)
    i3 = jnp.where(c, i, i3)
    c = a3 < a2
    a2, a3 = jnp.where(c, a3, a2), jnp.where(c, a2, a3)
    i2, i3 = jnp.where(c, i3, i2), jnp.where(c, i2, i3)
    c = a2 < a1
    a1, a2 = jnp.where(c, a2, a1), jnp.where(c, a1, a2)
    i1, i2 = jnp.where(c, i2, i1), jnp.where(c, i1, i2)
    c = a1 < a0
    a0, a1 = jnp.where(c, a1, a0), jnp.where(c, a0, a1)
    i0, i1 = jnp.where(c, i1, i0), jnp.where(c, i0, i1)
    return (a0, a1, a2, a3, i0, i1, i2, i3)


def _merge_and_gather(state, sco_v):
    a0, a1, a2, a3, i0, i1, i2, i3 = state
    lane = lax.iota(jnp.int32, LANES)
    inf = jnp.float32(jnp.inf)
    idxs = jnp.zeros((LANES,), jnp.int32)
    for k in range(4):
        tv = _bfly_min(a0)
        ti = _bfly_min(jnp.where(a0 == tv, i0, N))
        idxs = jnp.where(lane == k, ti, idxs)
        pop = (a0 == tv) & (i0 == ti)
        a0 = jnp.where(pop, a1, a0)
        i0 = jnp.where(pop, i1, i0)
        a1 = jnp.where(pop, a2, a1)
        i1 = jnp.where(pop, i2, i1)
        a2 = jnp.where(pop, a3, a2)
        i2 = jnp.where(pop, i3, i2)
        a3 = jnp.where(pop, inf, a3)
        i3 = jnp.where(pop, N, i3)
    g = plsc.load_gather(sco_v, [idxs])
    return _bfly_sum(jnp.where(lane < 4, g, 0.0))


def _sc_body(
    score_hbm, error_hbm, out_hbm, e0, e1, s0, s1, res_v, sm0, sm1, sm2, sm3
):
    wid = lax.axis_index("s") * NUM_CORES + lax.axis_index("c")
    r0 = wid * ROWS_PER_WORKER
    r1 = r0 + 1
    H = N // 4
    R = N - H
    ha0 = pltpu.async_copy(error_hbm.at[r0, pl.ds(0, H)], e0.at[pl.ds(0, H)], sm0)
    ha1 = pltpu.async_copy(error_hbm.at[r1, pl.ds(0, H)], e1.at[pl.ds(0, H)], sm1)
    hb0 = pltpu.async_copy(
        error_hbm.at[r0, pl.ds(H, R)], e0.at[pl.ds(H, R)], sm2
    )
    hb1 = pltpu.async_copy(
        error_hbm.at[r1, pl.ds(H, R)], e1.at[pl.ds(H, R)], sm3
    )
    ha0.wait()
    ha1.wait()
    hs0 = pltpu.async_copy(score_hbm.at[r0], s0, sm0)
    hs1 = pltpu.async_copy(score_hbm.at[r1], s1, sm1)

    lane = lax.iota(jnp.int32, LANES)
    inf = jnp.float32(jnp.inf)
    va = jnp.full((LANES,), inf, jnp.float32)
    vi = jnp.full((LANES,), N, jnp.int32)
    init = (va, va, va, va, vi, vi, vi, vi)

    def body(j, carry):
        st0, st1 = carry
        base = j * LANES
        i = base + lane
        st0 = _insert(st0, e0[pl.ds(base, LANES)], i)
        st1 = _insert(st1, e1[pl.ds(base, LANES)], i)
        return st0, st1

    st0, st1 = lax.fori_loop(0, NBLK // 4, body, (init, init))
    hb0.wait()
    hb1.wait()
    st0, st1 = lax.fori_loop(NBLK // 4, NBLK, body, (st0, st1))

    hs0.wait()
    hs1.wait()
    sum0 = _merge_and_gather(st0, s0)
    sum1 = _merge_and_gather(st1, s1)
    res = jnp.where(lane == 0, sum0, jnp.where(lane == 1, sum1, 0.0))
    res_v[...] = res
    pltpu.sync_copy(res_v, out_hbm.at[wid])


def _selected_sums(score, error):
    mesh = plsc.VectorSubcoreMesh(
        core_axis_name="c",
        subcore_axis_name="s",
        num_cores=NUM_CORES,
        num_subcores=NUM_SUBCORES,
    )
    k = pl.kernel(
        _sc_body,
        out_type=jax.ShapeDtypeStruct((NUM_WORKERS, LANES), jnp.float32),
        mesh=mesh,
        scratch_types=[
            pltpu.VMEM((N,), jnp.float32),
            pltpu.VMEM((N,), jnp.float32),
            pltpu.VMEM((N,), jnp.float32),
            pltpu.VMEM((N,), jnp.float32),
            pltpu.VMEM((LANES,), jnp.float32),
            pltpu.SemaphoreType.DMA,
            pltpu.SemaphoreType.DMA,
            pltpu.SemaphoreType.DMA,
            pltpu.SemaphoreType.DMA,
        ],
        compiler_params=pltpu.CompilerParams(
            needs_layout_passes=False, skip_device_barrier=True
        ),
    )
    return k(score, error)


def kernel(score, error):
    sp_sum = _softplus_sum(score)[0, 0]
    sel = _selected_sums(score, error)
    sel_sum = jnp.sum(sel[:, :ROWS_PER_WORKER])
    return (sp_sum - sel_sum) / (B * N)
